# baseline (speedup 1.0000x reference)
import functools

import jax
import jax.numpy as jnp
from jax import lax
from jax.experimental import pallas as pl
from jax.experimental.pallas import tpu as pltpu
from jax.experimental.pallas import tpu_sc as plsc

N_NODES = 50000
N_EDGES = 800000
NUM_RELS = 8
NUM_BASES = 4
H_DIM = 32
OUT_DIM = 8

NC, NS = 2, 16
NW = NC * NS
CHUNK = 96
KC = 264
E_PAD = NW * CHUNK * KC
SWC = 24
NSW = KC // SWC
G = 6
LOOKA = 3
N_PAD = 50048
NSL = N_PAD // NS
ZR = 136


def _lane_bcast(vec, e):
    return lax.gather(
        vec, jnp.full((16, 1), e, jnp.int32),
        dimension_numbers=lax.GatherDimensionNumbers(
            offset_dims=(), collapsed_slice_dims=(0,), start_index_map=(0,)),
        slice_sizes=(1,), mode=lax.GatherScatterMode.PROMISE_IN_BOUNDS)


def _sc_segment_sum(D):
    mesh = plsc.VectorSubcoreMesh(core_axis_name="c", subcore_axis_name="s")

    def body(table, idx1, dst1, norm1, out, acc, idxb, dstb, normb, zbuf,
             rows, gsem, ssem):
        c = lax.axis_index("c")
        s = lax.axis_index("s")
        wid = c * NS + s

        for i in range(ZR):
            for d0 in range(D // 16):
                zbuf[i, pl.ds(d0 * 16, 16)] = jnp.zeros((16,), jnp.float32)

        def zloop(i, carry):
            pltpu.sync_copy(zbuf, acc.at[pl.ds(s * NSL + i * ZR, ZR)])
            return carry
        lax.fori_loop(0, NSL // ZR, zloop, 0)

        plsc.subcore_barrier()

        def gather(j, b):
            pltpu.async_copy(table.at[idxb.at[pl.ds(j * CHUNK, CHUNK)]],
                             rows.at[b], gsem)

        def wait_gather(j, b):
            pltpu.make_async_copy(table.at[idxb.at[pl.ds(j * CHUNK, CHUNK)]],
                                  rows.at[b], gsem).wait()

        def scatter(j, b):
            pltpu.async_copy(rows.at[b],
                             acc.at[dstb.at[pl.ds(j * CHUNK, CHUNK)]],
                             ssem, add=True)

        def wait_scatter(j, b):
            pltpu.make_async_copy(rows.at[b],
                                  acc.at[dstb.at[pl.ds(j * CHUNK, CHUNK)]],
                                  ssem).wait()

        def swloop(w, carry):
            base = (wid * KC + w * SWC) * CHUNK
            pltpu.sync_copy(idx1.at[pl.ds(base, SWC * CHUNK)], idxb)
            pltpu.sync_copy(dst1.at[pl.ds(base, SWC * CHUNK)], dstb)
            pltpu.sync_copy(norm1.at[pl.ds(base, SWC * CHUNK)], normb)
            for p in range(LOOKA):
                gather(p, p)

            def quad(jj, carry2):
                for u in range(G):
                    j = G * jj + u
                    pl.when(j >= LOOKA)(
                        lambda: wait_scatter(j - LOOKA, (u - LOOKA) % G))
                    pl.when(j + LOOKA < SWC)(
                        lambda: gather(j + LOOKA, (u + LOOKA) % G))
                    wait_gather(j, u)
                    for g in range(CHUNK // 16):
                        nvec = normb[pl.ds(j * CHUNK + g * 16, 16)]
                        for e in range(16):
                            bc = _lane_bcast(nvec, e)
                            r_i = g * 16 + e
                            for d0 in range(D // 16):
                                sl = pl.ds(d0 * 16, 16)
                                rows[u, r_i, sl] = rows[u, r_i, sl] * bc
                    scatter(j, u)
                return carry2
            lax.fori_loop(0, SWC // G, quad, 0)
            for p in range(LOOKA):
                jt = SWC - LOOKA + p
                wait_scatter(jt, jt % G)
            return carry
        lax.fori_loop(0, NSW, swloop, 0)

        plsc.subcore_barrier()
        pltpu.sync_copy(acc.at[pl.ds(s * NSL, NSL)],
                        out.at[c, pl.ds(s * NSL, NSL)])

    return pl.kernel(
        body,
        out_type=jax.ShapeDtypeStruct((NC, N_PAD, D), jnp.float32),
        mesh=mesh,
        compiler_params=pltpu.CompilerParams(use_tc_tiling_on_sc=False),
        scratch_types=[
            pltpu.VMEM_SHARED((N_PAD, D), jnp.float32),
            pltpu.VMEM((SWC * CHUNK,), jnp.int32),
            pltpu.VMEM((SWC * CHUNK,), jnp.int32),
            pltpu.VMEM((SWC * CHUNK,), jnp.float32),
            pltpu.VMEM((ZR, D), jnp.float32),
            pltpu.VMEM((G, CHUNK, D), jnp.float32),
            pltpu.SemaphoreType.DMA,
            pltpu.SemaphoreType.DMA,
        ],
    )


BN = 2000


def _tc_embed(wv, m0):
    def body(wv_ref, m_ref, out_ref):
        out_ref[...] = jnp.dot(wv_ref[...], m_ref[...],
                               preferred_element_type=jnp.float32)
    return pl.pallas_call(
        body,
        grid=(N_NODES // BN,),
        in_specs=[pl.BlockSpec((BN, 128), lambda i: (i, 0)),
                  pl.BlockSpec((128, 256), lambda i: (0, 0))],
        out_specs=pl.BlockSpec((BN, 256), lambda i: (i, 0)),
        out_shape=jax.ShapeDtypeStruct((N_NODES, 256), jnp.float32),
    )(wv, m0)


def _tc_hw(acc, wcat):
    wdo = wcat.shape[1]
    def body(a0, a1, w_ref, out_ref):
        h = jnp.maximum(a0[0] + a1[0], 0.0)
        out_ref[...] = jnp.dot(h, w_ref[...],
                               preferred_element_type=jnp.float32)
    return pl.pallas_call(
        body,
        grid=(N_NODES // BN,),
        in_specs=[pl.BlockSpec((1, BN, H_DIM), lambda i: (0, i, 0)),
                  pl.BlockSpec((1, BN, H_DIM), lambda i: (1, i, 0)),
                  pl.BlockSpec((H_DIM, wdo), lambda i: (0, 0))],
        out_specs=pl.BlockSpec((BN, wdo), lambda i: (i, 0)),
        out_shape=jax.ShapeDtypeStruct((N_NODES, wdo), jnp.float32),
    )(acc, acc, wcat)


def _tc_softmax(acc):
    def body(a0, a1, out_ref):
        x = a0[0] + a1[0]
        lane = lax.broadcasted_iota(jnp.int32, (BN, 16), 1)
        xm = jnp.where(lane < OUT_DIM, x, -jnp.inf)
        m = jnp.max(xm, axis=1, keepdims=True)
        e = jnp.exp(xm - m)
        ssum = jnp.sum(e, axis=1, keepdims=True)
        out_ref[...] = (e / ssum)[:, :OUT_DIM]
    return pl.pallas_call(
        body,
        grid=(N_NODES // BN,),
        in_specs=[pl.BlockSpec((1, BN, 16), lambda i: (0, i, 0)),
                  pl.BlockSpec((1, BN, 16), lambda i: (1, i, 0))],
        out_specs=pl.BlockSpec((BN, OUT_DIM), lambda i: (i, 0)),
        out_shape=jax.ShapeDtypeStruct((N_NODES, OUT_DIM), jnp.float32),
    )(acc, acc)


def _mk_M(C, out):
    return jnp.reshape(
        jnp.transpose(C)[:, None, :, None] * jnp.eye(out, dtype=C.dtype)[None, :, None, :],
        (NUM_BASES * out, NUM_RELS * out))


def kernel(edge_src, edge_dst, edge_type, edge_norm,
           weight0, w_comp0, weight1, w_comp1, weight2, w_comp2):
    pad = E_PAD - N_EDGES
    ar = jnp.arange(pad, dtype=jnp.int32)
    src = edge_src.astype(jnp.int32)
    typ = edge_type.astype(jnp.int32)
    padv = ar % N_NODES
    idx0 = jnp.concatenate([typ * N_NODES + src, padv])
    idx12 = jnp.concatenate([src * NUM_RELS + typ, padv])
    dst = jnp.concatenate([edge_dst.astype(jnp.int32), padv])
    nrm = jnp.concatenate([edge_norm[:, 0], jnp.zeros((pad,), jnp.float32)])

    sc32 = _sc_segment_sum(H_DIM)
    sc16 = _sc_segment_sum(16)

    M0 = _mk_M(w_comp0, H_DIM)
    embed = _tc_embed(weight0.reshape(N_NODES, NUM_BASES * H_DIM), M0
                      ).reshape(NUM_RELS * N_NODES, H_DIM)
    acc = sc32(embed, idx0, dst, nrm)

    M1 = _mk_M(w_comp1, H_DIM)
    W1 = (weight1.reshape(H_DIM, NUM_BASES * H_DIM) @ M1
          ).reshape(NUM_RELS, H_DIM, H_DIM)
    wcat1 = jnp.transpose(W1, (1, 0, 2)).reshape(H_DIM, NUM_RELS * H_DIM)
    hw1 = _tc_hw(acc, wcat1).reshape(NUM_RELS * N_NODES, H_DIM)
    acc = sc32(hw1, idx12, dst, nrm)

    M2 = _mk_M(w_comp2, OUT_DIM)
    W2 = (weight2.reshape(H_DIM, NUM_BASES * OUT_DIM) @ M2
          ).reshape(NUM_RELS, H_DIM, OUT_DIM)
    W2p = jnp.pad(W2, ((0, 0), (0, 0), (0, 16 - OUT_DIM)))
    wcat2 = jnp.transpose(W2p, (1, 0, 2)).reshape(H_DIM, NUM_RELS * 16)
    hw2 = _tc_hw(acc, wcat2).reshape(NUM_RELS * N_NODES, 16)
    acc = sc16(hw2, idx12, dst, nrm)
    return _tc_softmax(acc)


# speedup vs baseline: 1.0913x; 1.0913x over previous
import functools

import jax
import jax.numpy as jnp
from jax import lax
from jax.experimental import pallas as pl
from jax.experimental.pallas import tpu as pltpu
from jax.experimental.pallas import tpu_sc as plsc

N_NODES = 50000
N_EDGES = 800000
NUM_RELS = 8
NUM_BASES = 4
H_DIM = 32
OUT_DIM = 8

NC, NS = 2, 16
NW = NC * NS
CHUNK = 128
KC = 200
E_PAD = NW * CHUNK * KC
SWC = 20
NSW = KC // SWC
G = 4
LOOKA = 2
N_PAD = 50048
NSL = N_PAD // NS
ZR = 136


def _lane_bcast(vec, e):
    return lax.gather(
        vec, jnp.full((16, 1), e, jnp.int32),
        dimension_numbers=lax.GatherDimensionNumbers(
            offset_dims=(), collapsed_slice_dims=(0,), start_index_map=(0,)),
        slice_sizes=(1,), mode=lax.GatherScatterMode.PROMISE_IN_BOUNDS)


def _sc_segment_sum(D):
    mesh = plsc.VectorSubcoreMesh(core_axis_name="c", subcore_axis_name="s")

    def body(table, idx1, dst1, norm1, out, acc, idxb, dstb, normb, zbuf,
             rows, gsem, ssem):
        c = lax.axis_index("c")
        s = lax.axis_index("s")
        wid = c * NS + s

        for i in range(ZR):
            for d0 in range(D // 16):
                zbuf[i, pl.ds(d0 * 16, 16)] = jnp.zeros((16,), jnp.float32)

        def zloop(i, carry):
            pltpu.sync_copy(zbuf, acc.at[pl.ds(s * NSL + i * ZR, ZR)])
            return carry
        lax.fori_loop(0, NSL // ZR, zloop, 0)

        plsc.subcore_barrier()

        def gather(j, b):
            pltpu.async_copy(table.at[idxb.at[pl.ds(j * CHUNK, CHUNK)]],
                             rows.at[b], gsem)

        def wait_gather(j, b):
            pltpu.make_async_copy(table.at[idxb.at[pl.ds(j * CHUNK, CHUNK)]],
                                  rows.at[b], gsem).wait()

        def scatter(j, b):
            pltpu.async_copy(rows.at[b],
                             acc.at[dstb.at[pl.ds(j * CHUNK, CHUNK)]],
                             ssem, add=True)

        def wait_scatter(j, b):
            pltpu.make_async_copy(rows.at[b],
                                  acc.at[dstb.at[pl.ds(j * CHUNK, CHUNK)]],
                                  ssem).wait()

        def swloop(w, carry):
            base = (wid * KC + w * SWC) * CHUNK
            pltpu.sync_copy(idx1.at[pl.ds(base, SWC * CHUNK)], idxb)
            pltpu.sync_copy(dst1.at[pl.ds(base, SWC * CHUNK)], dstb)
            pltpu.sync_copy(norm1.at[pl.ds(base, SWC * CHUNK)], normb)
            for p in range(LOOKA):
                gather(p, p)

            def quad(jj, carry2):
                for u in range(G):
                    j = G * jj + u
                    pl.when(j >= LOOKA)(
                        lambda: wait_scatter(j - LOOKA, (u - LOOKA) % G))
                    pl.when(j + LOOKA < SWC)(
                        lambda: gather(j + LOOKA, (u + LOOKA) % G))
                    wait_gather(j, u)
                    for g in range(CHUNK // 16):
                        nvec = normb[pl.ds(j * CHUNK + g * 16, 16)]
                        for e in range(16):
                            bc = _lane_bcast(nvec, e)
                            r_i = g * 16 + e
                            for d0 in range(D // 16):
                                sl = pl.ds(d0 * 16, 16)
                                rows[u, r_i, sl] = rows[u, r_i, sl] * bc
                    scatter(j, u)
                return carry2
            lax.fori_loop(0, SWC // G, quad, 0)
            for p in range(LOOKA):
                jt = SWC - LOOKA + p
                wait_scatter(jt, jt % G)
            return carry
        lax.fori_loop(0, NSW, swloop, 0)

        plsc.subcore_barrier()
        pltpu.sync_copy(acc.at[pl.ds(s * NSL, NSL)],
                        out.at[c, pl.ds(s * NSL, NSL)])

    return pl.kernel(
        body,
        out_type=jax.ShapeDtypeStruct((NC, N_PAD, D), jnp.float32),
        mesh=mesh,
        compiler_params=pltpu.CompilerParams(use_tc_tiling_on_sc=False),
        scratch_types=[
            pltpu.VMEM_SHARED((N_PAD, D), jnp.float32),
            pltpu.VMEM((SWC * CHUNK,), jnp.int32),
            pltpu.VMEM((SWC * CHUNK,), jnp.int32),
            pltpu.VMEM((SWC * CHUNK,), jnp.float32),
            pltpu.VMEM((ZR, D), jnp.float32),
            pltpu.VMEM((G, CHUNK, D), jnp.float32),
            pltpu.SemaphoreType.DMA,
            pltpu.SemaphoreType.DMA,
        ],
    )


BN = 2000


def _tc_embed(w2, m0):
    def body(w_ref, m_ref, out_ref):
        out_ref[...] = jnp.dot(w_ref[...], m_ref[...],
                               preferred_element_type=jnp.float32)
    return pl.pallas_call(
        body,
        grid=(N_NODES // BN,),
        in_specs=[pl.BlockSpec((BN, NUM_BASES * H_DIM), lambda i: (i, 0)),
                  pl.BlockSpec((NUM_BASES * H_DIM, NUM_RELS * H_DIM),
                               lambda i: (0, 0))],
        out_specs=pl.BlockSpec((BN, NUM_RELS * H_DIM), lambda i: (i, 0)),
        out_shape=jax.ShapeDtypeStruct((N_NODES, NUM_RELS * H_DIM),
                                       jnp.float32),
    )(w2, m0)


def _tc_hw(acc, wcat):
    wdo = wcat.shape[1]
    def body(a0, a1, w_ref, out_ref):
        h = jnp.maximum(a0[0] + a1[0], 0.0)
        out_ref[...] = jnp.dot(h, w_ref[...],
                               preferred_element_type=jnp.float32)
    return pl.pallas_call(
        body,
        grid=(N_NODES // BN,),
        in_specs=[pl.BlockSpec((1, BN, H_DIM), lambda i: (0, i, 0)),
                  pl.BlockSpec((1, BN, H_DIM), lambda i: (1, i, 0)),
                  pl.BlockSpec((H_DIM, wdo), lambda i: (0, 0))],
        out_specs=pl.BlockSpec((BN, wdo), lambda i: (i, 0)),
        out_shape=jax.ShapeDtypeStruct((N_NODES, wdo), jnp.float32),
    )(acc, acc, wcat)


def _tc_softmax(acc):
    def body(a0, a1, out_ref):
        x = a0[0] + a1[0]
        lane = lax.broadcasted_iota(jnp.int32, (BN, 16), 1)
        xm = jnp.where(lane < OUT_DIM, x, -jnp.inf)
        m = jnp.max(xm, axis=1, keepdims=True)
        e = jnp.exp(xm - m)
        ssum = jnp.sum(e, axis=1, keepdims=True)
        out_ref[...] = (e / ssum)[:, :OUT_DIM]
    return pl.pallas_call(
        body,
        grid=(N_NODES // BN,),
        in_specs=[pl.BlockSpec((1, BN, 16), lambda i: (0, i, 0)),
                  pl.BlockSpec((1, BN, 16), lambda i: (1, i, 0))],
        out_specs=pl.BlockSpec((BN, OUT_DIM), lambda i: (i, 0)),
        out_shape=jax.ShapeDtypeStruct((N_NODES, OUT_DIM), jnp.float32),
    )(acc, acc)


def _mk_M(C, out):
    return jnp.reshape(
        jnp.transpose(C)[:, None, :, None] * jnp.eye(out, dtype=C.dtype)[None, :, None, :],
        (NUM_BASES * out, NUM_RELS * out))


def kernel(edge_src, edge_dst, edge_type, edge_norm,
           weight0, w_comp0, weight1, w_comp1, weight2, w_comp2):
    pad = E_PAD - N_EDGES
    ar = jnp.arange(pad, dtype=jnp.int32)
    src = edge_src.astype(jnp.int32)
    typ = edge_type.astype(jnp.int32)
    padv = ar % N_NODES
    idx0 = jnp.concatenate([typ * N_NODES + src, padv])
    idx12 = jnp.concatenate([src * NUM_RELS + typ, padv])
    dst = jnp.concatenate([edge_dst.astype(jnp.int32), padv])
    nrm = jnp.concatenate([edge_norm[:, 0], jnp.zeros((pad,), jnp.float32)])

    sc32 = _sc_segment_sum(H_DIM)
    sc16 = _sc_segment_sum(16)

    M0 = _mk_M(w_comp0, H_DIM)
    embed = _tc_embed(weight0.reshape(N_NODES, NUM_BASES * H_DIM), M0
                      ).reshape(NUM_RELS * N_NODES, H_DIM)
    acc = sc32(embed, idx0, dst, nrm)

    M1 = _mk_M(w_comp1, H_DIM)
    W1 = (weight1.reshape(H_DIM, NUM_BASES * H_DIM) @ M1
          ).reshape(NUM_RELS, H_DIM, H_DIM)
    wcat1 = jnp.transpose(W1, (1, 0, 2)).reshape(H_DIM, NUM_RELS * H_DIM)
    hw1 = _tc_hw(acc, wcat1).reshape(NUM_RELS * N_NODES, H_DIM)
    acc = sc32(hw1, idx12, dst, nrm)

    M2 = _mk_M(w_comp2, OUT_DIM)
    W2 = (weight2.reshape(H_DIM, NUM_BASES * OUT_DIM) @ M2
          ).reshape(NUM_RELS, H_DIM, OUT_DIM)
    W2p = jnp.pad(W2, ((0, 0), (0, 0), (0, 16 - OUT_DIM)))
    wcat2 = jnp.transpose(W2p, (1, 0, 2)).reshape(H_DIM, NUM_RELS * 16)
    hw2 = _tc_hw(acc, wcat2).reshape(NUM_RELS * N_NODES, 16)
    acc = sc16(hw2, idx12, dst, nrm)
    return _tc_softmax(acc)


# speedup vs baseline: 1.1020x; 1.0099x over previous
import functools

import jax
import jax.numpy as jnp
from jax import lax
from jax.experimental import pallas as pl
from jax.experimental.pallas import tpu as pltpu
from jax.experimental.pallas import tpu_sc as plsc

N_NODES = 50000
N_EDGES = 800000
NUM_RELS = 8
NUM_BASES = 4
H_DIM = 32
OUT_DIM = 8

NC, NS = 2, 16
NW = NC * NS
CHUNK = 112
KC = 224
E_PAD = NW * CHUNK * KC
SWC = 28
NSW = KC // SWC
G = 4
LOOKA = 2
N_PAD = 50048
NSL = N_PAD // NS
ZR = 136


def _lane_bcast(vec, e):
    return lax.gather(
        vec, jnp.full((16, 1), e, jnp.int32),
        dimension_numbers=lax.GatherDimensionNumbers(
            offset_dims=(), collapsed_slice_dims=(0,), start_index_map=(0,)),
        slice_sizes=(1,), mode=lax.GatherScatterMode.PROMISE_IN_BOUNDS)


def _sc_segment_sum(D):
    mesh = plsc.VectorSubcoreMesh(core_axis_name="c", subcore_axis_name="s")

    def body(table, idx1, dst1, norm1, out, acc, idxb, dstb, normb, zbuf,
             rows, gsem, ssem, zsem):
        c = lax.axis_index("c")
        s = lax.axis_index("s")
        wid = c * NS + s

        for i in range(ZR):
            for d0 in range(D // 16):
                zbuf[i, pl.ds(d0 * 16, 16)] = jnp.zeros((16,), jnp.float32)

        def zloop(i, carry):
            pltpu.async_copy(zbuf, acc.at[pl.ds(s * NSL + i * ZR, ZR)],
                             zsem)
            return carry
        lax.fori_loop(0, NSL // ZR, zloop, 0)

        def zdrain(i, carry):
            pltpu.make_async_copy(zbuf,
                                  acc.at[pl.ds(s * NSL + i * ZR, ZR)],
                                  zsem).wait()
            return carry
        lax.fori_loop(0, NSL // ZR, zdrain, 0)
        plsc.subcore_barrier()

        def gather(j, b):
            pltpu.async_copy(table.at[idxb.at[pl.ds(j * CHUNK, CHUNK)]],
                             rows.at[b], gsem)

        def wait_gather(j, b):
            pltpu.make_async_copy(table.at[idxb.at[pl.ds(j * CHUNK, CHUNK)]],
                                  rows.at[b], gsem).wait()

        def scatter(j, b):
            pltpu.async_copy(rows.at[b],
                             acc.at[dstb.at[pl.ds(j * CHUNK, CHUNK)]],
                             ssem, add=True)

        def wait_scatter(j, b):
            pltpu.make_async_copy(rows.at[b],
                                  acc.at[dstb.at[pl.ds(j * CHUNK, CHUNK)]],
                                  ssem).wait()

        def swloop(w, carry):
            base = (wid * KC + w * SWC) * CHUNK
            pltpu.sync_copy(idx1.at[pl.ds(base, SWC * CHUNK)], idxb)
            pltpu.sync_copy(dst1.at[pl.ds(base, SWC * CHUNK)], dstb)
            pltpu.sync_copy(norm1.at[pl.ds(base, SWC * CHUNK)], normb)
            for p in range(LOOKA):
                gather(p, p)

            def quad(jj, carry2):
                for u in range(G):
                    j = G * jj + u
                    pl.when(j >= LOOKA)(
                        lambda: wait_scatter(j - LOOKA, (u - LOOKA) % G))
                    pl.when(j + LOOKA < SWC)(
                        lambda: gather(j + LOOKA, (u + LOOKA) % G))
                    wait_gather(j, u)
                    for g in range(CHUNK // 16):
                        nvec = normb[pl.ds(j * CHUNK + g * 16, 16)]
                        for e in range(16):
                            bc = _lane_bcast(nvec, e)
                            r_i = g * 16 + e
                            for d0 in range(D // 16):
                                sl = pl.ds(d0 * 16, 16)
                                rows[u, r_i, sl] = rows[u, r_i, sl] * bc
                    scatter(j, u)
                return carry2
            lax.fori_loop(0, SWC // G, quad, 0)
            for p in range(LOOKA):
                jt = SWC - LOOKA + p
                wait_scatter(jt, jt % G)
            return carry
        lax.fori_loop(0, NSW, swloop, 0)

        plsc.subcore_barrier()
        pltpu.sync_copy(acc.at[pl.ds(s * NSL, NSL)],
                        out.at[c, pl.ds(s * NSL, NSL)])

    return pl.kernel(
        body,
        out_type=jax.ShapeDtypeStruct((NC, N_PAD, D), jnp.float32),
        mesh=mesh,
        compiler_params=pltpu.CompilerParams(use_tc_tiling_on_sc=False),
        scratch_types=[
            pltpu.VMEM_SHARED((N_PAD, D), jnp.float32),
            pltpu.VMEM((SWC * CHUNK,), jnp.int32),
            pltpu.VMEM((SWC * CHUNK,), jnp.int32),
            pltpu.VMEM((SWC * CHUNK,), jnp.float32),
            pltpu.VMEM((ZR, D), jnp.float32),
            pltpu.VMEM((G, CHUNK, D), jnp.float32),
            pltpu.SemaphoreType.DMA,
            pltpu.SemaphoreType.DMA,
            pltpu.SemaphoreType.DMA,
        ],
    )


BN = 2000


def _tc_embed(w2, m0):
    def body(w_ref, m_ref, out_ref):
        out_ref[...] = jnp.dot(w_ref[...], m_ref[...],
                               preferred_element_type=jnp.float32)
    return pl.pallas_call(
        body,
        grid=(N_NODES // BN,),
        in_specs=[pl.BlockSpec((BN, NUM_BASES * H_DIM), lambda i: (i, 0)),
                  pl.BlockSpec((NUM_BASES * H_DIM, NUM_RELS * H_DIM),
                               lambda i: (0, 0))],
        out_specs=pl.BlockSpec((BN, NUM_RELS * H_DIM), lambda i: (i, 0)),
        out_shape=jax.ShapeDtypeStruct((N_NODES, NUM_RELS * H_DIM),
                                       jnp.float32),
    )(w2, m0)


def _tc_hw(acc, wcat):
    wdo = wcat.shape[1]
    def body(a0, a1, w_ref, out_ref):
        h = jnp.maximum(a0[0] + a1[0], 0.0)
        out_ref[...] = jnp.dot(h, w_ref[...],
                               preferred_element_type=jnp.float32)
    return pl.pallas_call(
        body,
        grid=(N_NODES // BN,),
        in_specs=[pl.BlockSpec((1, BN, H_DIM), lambda i: (0, i, 0)),
                  pl.BlockSpec((1, BN, H_DIM), lambda i: (1, i, 0)),
                  pl.BlockSpec((H_DIM, wdo), lambda i: (0, 0))],
        out_specs=pl.BlockSpec((BN, wdo), lambda i: (i, 0)),
        out_shape=jax.ShapeDtypeStruct((N_NODES, wdo), jnp.float32),
    )(acc, acc, wcat)


def _tc_softmax(acc):
    def body(a0, a1, out_ref):
        x = a0[0] + a1[0]
        lane = lax.broadcasted_iota(jnp.int32, (BN, 16), 1)
        xm = jnp.where(lane < OUT_DIM, x, -jnp.inf)
        m = jnp.max(xm, axis=1, keepdims=True)
        e = jnp.exp(xm - m)
        ssum = jnp.sum(e, axis=1, keepdims=True)
        out_ref[...] = (e / ssum)[:, :OUT_DIM]
    return pl.pallas_call(
        body,
        grid=(N_NODES // BN,),
        in_specs=[pl.BlockSpec((1, BN, 16), lambda i: (0, i, 0)),
                  pl.BlockSpec((1, BN, 16), lambda i: (1, i, 0))],
        out_specs=pl.BlockSpec((BN, OUT_DIM), lambda i: (i, 0)),
        out_shape=jax.ShapeDtypeStruct((N_NODES, OUT_DIM), jnp.float32),
    )(acc, acc)


def _mk_M(C, out):
    return jnp.reshape(
        jnp.transpose(C)[:, None, :, None] * jnp.eye(out, dtype=C.dtype)[None, :, None, :],
        (NUM_BASES * out, NUM_RELS * out))


def kernel(edge_src, edge_dst, edge_type, edge_norm,
           weight0, w_comp0, weight1, w_comp1, weight2, w_comp2):
    pad = E_PAD - N_EDGES
    ar = jnp.arange(pad, dtype=jnp.int32)
    src = edge_src.astype(jnp.int32)
    typ = edge_type.astype(jnp.int32)
    padv = ar % N_NODES
    idx0 = jnp.concatenate([typ * N_NODES + src, padv])
    idx12 = jnp.concatenate([src * NUM_RELS + typ, padv])
    dst = jnp.concatenate([edge_dst.astype(jnp.int32), padv])
    nrm = jnp.concatenate([edge_norm[:, 0], jnp.zeros((pad,), jnp.float32)])

    sc32 = _sc_segment_sum(H_DIM)
    sc16 = _sc_segment_sum(16)

    M0 = _mk_M(w_comp0, H_DIM)
    embed = _tc_embed(weight0.reshape(N_NODES, NUM_BASES * H_DIM), M0
                      ).reshape(NUM_RELS * N_NODES, H_DIM)
    acc = sc32(embed, idx0, dst, nrm)

    M1 = _mk_M(w_comp1, H_DIM)
    W1 = (weight1.reshape(H_DIM, NUM_BASES * H_DIM) @ M1
          ).reshape(NUM_RELS, H_DIM, H_DIM)
    wcat1 = jnp.transpose(W1, (1, 0, 2)).reshape(H_DIM, NUM_RELS * H_DIM)
    hw1 = _tc_hw(acc, wcat1).reshape(NUM_RELS * N_NODES, H_DIM)
    acc = sc32(hw1, idx12, dst, nrm)

    M2 = _mk_M(w_comp2, OUT_DIM)
    W2 = (weight2.reshape(H_DIM, NUM_BASES * OUT_DIM) @ M2
          ).reshape(NUM_RELS, H_DIM, OUT_DIM)
    W2p = jnp.pad(W2, ((0, 0), (0, 0), (0, 16 - OUT_DIM)))
    wcat2 = jnp.transpose(W2p, (1, 0, 2)).reshape(H_DIM, NUM_RELS * 16)
    hw2 = _tc_hw(acc, wcat2).reshape(NUM_RELS * N_NODES, 16)
    acc = sc16(hw2, idx12, dst, nrm)
    return _tc_softmax(acc)


# speedup vs baseline: 1.1035x; 1.0014x over previous
import jax
import jax.numpy as jnp
from jax import lax
from jax.experimental import pallas as pl
from jax.experimental.pallas import tpu as pltpu
from jax.experimental.pallas import tpu_sc as plsc

N_NODES = 50000
N_EDGES = 800000
NUM_RELS = 8
NUM_BASES = 4
H_DIM = 32
OUT_DIM = 8

NC, NS = 2, 16
NW = NC * NS
CHUNK = 112
KC = 224
E_PAD = NW * CHUNK * KC
SWC = 28
NSW = KC // SWC
G = 4
LOOKA = 2
N_PAD = 50048
NSL = N_PAD // NS
ZR = 136


def _lane_bcast(vec, e):
    return lax.gather(
        vec, jnp.full((16, 1), e, jnp.int32),
        dimension_numbers=lax.GatherDimensionNumbers(
            offset_dims=(), collapsed_slice_dims=(0,), start_index_map=(0,)),
        slice_sizes=(1,), mode=lax.GatherScatterMode.PROMISE_IN_BOUNDS)


def _sc_segment_sum(D):
    mesh = plsc.VectorSubcoreMesh(core_axis_name="c", subcore_axis_name="s")

    def body(table, idx1, dst1, norm1, out, acc, idxb, dstb, normb, zbuf,
             rows, gsem, ssem, zsem):
        c = lax.axis_index("c")
        s = lax.axis_index("s")
        wid = c * NS + s

        for i in range(ZR):
            for d0 in range(D // 16):
                zbuf[i, pl.ds(d0 * 16, 16)] = jnp.zeros((16,), jnp.float32)

        def zloop(i, carry):
            pltpu.async_copy(zbuf, acc.at[pl.ds(s * NSL + i * ZR, ZR)],
                             zsem)
            return carry
        lax.fori_loop(0, NSL // ZR, zloop, 0)

        def zdrain(i, carry):
            pltpu.make_async_copy(zbuf,
                                  acc.at[pl.ds(s * NSL + i * ZR, ZR)],
                                  zsem).wait()
            return carry
        lax.fori_loop(0, NSL // ZR, zdrain, 0)
        plsc.subcore_barrier()

        def gather(j, b):
            pltpu.async_copy(table.at[idxb.at[pl.ds(j * CHUNK, CHUNK)]],
                             rows.at[b], gsem)

        def wait_gather(j, b):
            pltpu.make_async_copy(table.at[idxb.at[pl.ds(j * CHUNK, CHUNK)]],
                                  rows.at[b], gsem).wait()

        def scatter(j, b):
            pltpu.async_copy(rows.at[b],
                             acc.at[dstb.at[pl.ds(j * CHUNK, CHUNK)]],
                             ssem, add=True)

        def wait_scatter(j, b):
            pltpu.make_async_copy(rows.at[b],
                                  acc.at[dstb.at[pl.ds(j * CHUNK, CHUNK)]],
                                  ssem).wait()

        def swloop(w, carry):
            base = (wid * KC + w * SWC) * CHUNK
            pltpu.sync_copy(idx1.at[pl.ds(base, SWC * CHUNK)], idxb)
            pltpu.sync_copy(dst1.at[pl.ds(base, SWC * CHUNK)], dstb)
            pltpu.sync_copy(norm1.at[pl.ds(base, SWC * CHUNK)], normb)
            for p in range(LOOKA):
                gather(p, p)

            def quad(jj, carry2):
                for u in range(G):
                    j = G * jj + u
                    pl.when(j >= LOOKA)(
                        lambda: wait_scatter(j - LOOKA, (u - LOOKA) % G))
                    pl.when(j + LOOKA < SWC)(
                        lambda: gather(j + LOOKA, (u + LOOKA) % G))
                    wait_gather(j, u)
                    for g in range(CHUNK // 16):
                        nvec = normb[pl.ds(j * CHUNK + g * 16, 16)]
                        for e in range(16):
                            bc = _lane_bcast(nvec, e)
                            r_i = g * 16 + e
                            for d0 in range(D // 16):
                                sl = pl.ds(d0 * 16, 16)
                                rows[u, r_i, sl] = rows[u, r_i, sl] * bc
                    scatter(j, u)
                return carry2
            lax.fori_loop(0, SWC // G, quad, 0)
            for p in range(LOOKA):
                jt = SWC - LOOKA + p
                wait_scatter(jt, jt % G)
            return carry
        lax.fori_loop(0, NSW, swloop, 0)

        plsc.subcore_barrier()
        pltpu.sync_copy(acc.at[pl.ds(s * NSL, NSL)],
                        out.at[c, pl.ds(s * NSL, NSL)])

    return pl.kernel(
        body,
        out_type=jax.ShapeDtypeStruct((NC, N_PAD, D), jnp.float32),
        mesh=mesh,
        compiler_params=pltpu.CompilerParams(use_tc_tiling_on_sc=False),
        scratch_types=[
            pltpu.VMEM_SHARED((N_PAD, D), jnp.float32),
            pltpu.VMEM((SWC * CHUNK,), jnp.int32),
            pltpu.VMEM((SWC * CHUNK,), jnp.int32),
            pltpu.VMEM((SWC * CHUNK,), jnp.float32),
            pltpu.VMEM((ZR, D), jnp.float32),
            pltpu.VMEM((G, CHUNK, D), jnp.float32),
            pltpu.SemaphoreType.DMA,
            pltpu.SemaphoreType.DMA,
            pltpu.SemaphoreType.DMA,
        ],
    )


BN = 2000


def _tc_embed(w2, m0):
    def body(w_ref, m_ref, out_ref):
        out_ref[...] = jnp.dot(w_ref[...], m_ref[...],
                               preferred_element_type=jnp.float32)
    return pl.pallas_call(
        body,
        grid=(N_NODES // BN,),
        in_specs=[pl.BlockSpec((BN, NUM_BASES * H_DIM), lambda i: (i, 0)),
                  pl.BlockSpec((NUM_BASES * H_DIM, NUM_RELS * H_DIM),
                               lambda i: (0, 0))],
        out_specs=pl.BlockSpec((BN, NUM_RELS * H_DIM), lambda i: (i, 0)),
        out_shape=jax.ShapeDtypeStruct((N_NODES, NUM_RELS * H_DIM),
                                       jnp.float32),
    )(w2, m0)


def _tc_hw(acc, wcat):
    wdo = wcat.shape[1]
    def body(a0, a1, w_ref, out_ref):
        h = jnp.maximum(a0[0] + a1[0], 0.0)
        out_ref[...] = jnp.dot(h, w_ref[...],
                               preferred_element_type=jnp.float32)
    return pl.pallas_call(
        body,
        grid=(N_NODES // BN,),
        in_specs=[pl.BlockSpec((1, BN, H_DIM), lambda i: (0, i, 0)),
                  pl.BlockSpec((1, BN, H_DIM), lambda i: (1, i, 0)),
                  pl.BlockSpec((H_DIM, wdo), lambda i: (0, 0))],
        out_specs=pl.BlockSpec((BN, wdo), lambda i: (i, 0)),
        out_shape=jax.ShapeDtypeStruct((N_NODES, wdo), jnp.float32),
    )(acc, acc, wcat)


def _tc_softmax(acc):
    def body(a0, a1, out_ref):
        x = a0[0] + a1[0]
        lane = lax.broadcasted_iota(jnp.int32, (BN, 16), 1)
        xm = jnp.where(lane < OUT_DIM, x, -jnp.inf)
        m = jnp.max(xm, axis=1, keepdims=True)
        e = jnp.exp(xm - m)
        ssum = jnp.sum(e, axis=1, keepdims=True)
        out_ref[...] = (e / ssum)[:, :OUT_DIM]
    return pl.pallas_call(
        body,
        grid=(N_NODES // BN,),
        in_specs=[pl.BlockSpec((1, BN, 16), lambda i: (0, i, 0)),
                  pl.BlockSpec((1, BN, 16), lambda i: (1, i, 0))],
        out_specs=pl.BlockSpec((BN, OUT_DIM), lambda i: (i, 0)),
        out_shape=jax.ShapeDtypeStruct((N_NODES, OUT_DIM), jnp.float32),
    )(acc, acc)


def _mk_M(C, out):
    return jnp.reshape(
        jnp.transpose(C)[:, None, :, None] * jnp.eye(out, dtype=C.dtype)[None, :, None, :],
        (NUM_BASES * out, NUM_RELS * out))


def kernel(edge_src, edge_dst, edge_type, edge_norm,
           weight0, w_comp0, weight1, w_comp1, weight2, w_comp2):
    pad = E_PAD - N_EDGES
    ar = jnp.arange(pad, dtype=jnp.int32)
    src = edge_src.astype(jnp.int32)
    typ = edge_type.astype(jnp.int32)
    padv = ar % N_NODES
    idx0 = jnp.concatenate([typ * N_NODES + src, padv])
    idx12 = jnp.concatenate([src * NUM_RELS + typ, padv])
    dst = jnp.concatenate([edge_dst.astype(jnp.int32), padv])
    nrm = jnp.concatenate([edge_norm[:, 0], jnp.zeros((pad,), jnp.float32)])

    sc32 = _sc_segment_sum(H_DIM)
    sc16 = _sc_segment_sum(16)

    M0 = _mk_M(w_comp0, H_DIM)
    embed = _tc_embed(weight0.reshape(N_NODES, NUM_BASES * H_DIM), M0
                      ).reshape(NUM_RELS * N_NODES, H_DIM)
    acc = sc32(embed, idx0, dst, nrm)

    M1 = _mk_M(w_comp1, H_DIM)
    W1 = (weight1.reshape(H_DIM, NUM_BASES * H_DIM) @ M1
          ).reshape(NUM_RELS, H_DIM, H_DIM)
    wcat1 = jnp.transpose(W1, (1, 0, 2)).reshape(H_DIM, NUM_RELS * H_DIM)
    hw1 = _tc_hw(acc, wcat1).reshape(NUM_RELS * N_NODES, H_DIM)
    acc = sc32(hw1, idx12, dst, nrm)

    M2 = _mk_M(w_comp2, OUT_DIM)
    W2 = (weight2.reshape(H_DIM, NUM_BASES * OUT_DIM) @ M2
          ).reshape(NUM_RELS, H_DIM, OUT_DIM)
    W2p = jnp.pad(W2, ((0, 0), (0, 0), (0, 16 - OUT_DIM)))
    wcat2 = jnp.transpose(W2p, (1, 0, 2)).reshape(H_DIM, NUM_RELS * 16)
    hw2 = _tc_hw(acc, wcat2).reshape(NUM_RELS * N_NODES, 16)
    acc = sc16(hw2, idx12, dst, nrm)
    return _tc_softmax(acc)
